# baseline (speedup 1.0000x reference)
import functools

import jax
import jax.numpy as jnp
from jax import lax
from jax.experimental import pallas as pl
from jax.experimental.pallas import tpu as pltpu
from jax.experimental.pallas import tpu_sc as plsc

D = 64
SCALE = 8.0
V = 1000000
B = 4096
S = 200

W1 = 128
NFULL = V // W1
VTAIL = V - NFULL * W1

NBT = B // 128
NST = S // 8

NRING1 = 4
NRING2 = 3


def _iota16():
    return lax.iota(jnp.int32, 16)


def _bcast16(v):
    return jnp.zeros((16,), jnp.int32) + v


@jax.jit
def _stage1(tbl_t, x_t):
    info = plsc.get_sparse_core_info()
    nw = info.num_cores * info.num_subcores
    mesh = plsc.VectorSubcoreMesh(core_axis_name="c", subcore_axis_name="s")
    n_xt = NST * NBT
    n_sup = n_xt // 4

    @functools.partial(
        pl.kernel,
        mesh=mesh,
        out_type=(
            jax.ShapeDtypeStruct((V * D,), jnp.float32),
            jax.ShapeDtypeStruct((n_xt, 8, 128), jnp.int32),
        ),
        scratch_types=(
            [pltpu.VMEM((D, W1), jnp.float32) for _ in range(NRING1)]
            + [pltpu.VMEM((W1 * D,), jnp.float32) for _ in range(NRING1)]
            + [
                pltpu.VMEM((D, VTAIL), jnp.float32),
                pltpu.VMEM((VTAIL * D,), jnp.float32),
                pltpu.VMEM((8, 512), jnp.int32),
                pltpu.VMEM((8, 512), jnp.int32),
            ]
            + [pltpu.SemaphoreType.DMA for _ in range(2 * NRING1 + 2)]
        ),
        compiler_params=pltpu.CompilerParams(needs_layout_passes=False),
    )
    def k1(tbl_hbm, x_hbm, tflat_hbm, idx_hbm, *bufs):
        sb = bufs[:NRING1]
        db = bufs[NRING1:2 * NRING1]
        st_v, dt_v, xa, xb = bufs[2 * NRING1:2 * NRING1 + 4]
        si = bufs[2 * NRING1 + 4:3 * NRING1 + 4]
        so = bufs[3 * NRING1 + 4:4 * NRING1 + 4]
        xsem, wsem = bufs[4 * NRING1 + 4:]

        wid = lax.axis_index("s") * info.num_cores + lax.axis_index("c")

        nb = NFULL // nw + jnp.where(wid < NFULL % nw, 1, 0)

        def blk_of(i):
            return wid + i * nw

        def start_in(i, r):
            pltpu.async_copy(
                tbl_hbm.at[:, pl.ds(blk_of(i) * W1, W1)], sb[r], si[r])

        def wait_in(r):
            pltpu.make_async_copy(
                tbl_hbm.at[:, pl.ds(0, W1)], sb[r], si[r]).wait()

        def start_out(i, r):
            pltpu.async_copy(
                db[r], tflat_hbm.at[pl.ds(blk_of(i) * W1 * D, W1 * D)],
                so[r])

        def wait_out(r):
            pltpu.make_async_copy(
                db[r], tflat_hbm.at[pl.ds(0, W1 * D)], so[r]).wait()

        for r in range(NRING1):
            @pl.when(nb > r)
            def _(r=r):
                start_in(r, r)

        n_x = n_sup // nw + jnp.where(wid < n_sup % nw, 1, 0)

        def sup_of(t):
            return wid + t * nw

        def x_read(t, buf):
            sup = sup_of(t)
            st = sup // (NBT // 4)
            bc4 = sup % (NBT // 4)
            pltpu.async_copy(
                x_hbm.at[pl.ds(st * 8, 8), pl.ds(bc4 * 512, 512)], buf, xsem)

        def x_step(t, buf):
            pltpu.make_async_copy(
                x_hbm.at[pl.ds(0, 8), pl.ds(0, 512)], buf, xsem).wait()
            tile0 = sup_of(t) * 4
            for k in range(4):
                pltpu.async_copy(
                    buf.at[:, pl.ds(k * 128, 128)],
                    idx_hbm.at[tile0 + k], wsem).wait()

            @pl.when(t + 2 < n_x)
            def _():
                x_read(t + 2, buf)

        x_read(0, xa)

        @pl.when(n_x > 1)
        def _():
            x_read(1, xb)

        def x_loop(t, carry):
            @pl.when(t < n_x)
            def _():
                @pl.when(t % 2 == 0)
                def _():
                    x_step(t, xa)

                @pl.when(t % 2 == 1)
                def _():
                    x_step(t, xb)

            return carry

        lax.fori_loop(0, n_sup // nw + 1, x_loop, 0)

        iota = _iota16()
        rowc = tuple(fc * 16 + iota for fc in range(D // 16))

        def transpose(sbuf, dbuf, width):
            def tbody(vl, colv):
                vals = [plsc.load_gather(sbuf, [rowc[fc], colv])
                        for fc in range(D // 16)]
                off = vl * D
                for fc in range(D // 16):
                    dbuf[pl.ds(off + fc * 16, 16)] = vals[fc] * SCALE
                return colv + 1

            lax.fori_loop(0, width, tbody, jnp.zeros((16,), jnp.int32),
                          unroll=4)

        def stage_step(i, r):
            wait_in(r)

            @pl.when(i >= NRING1)
            def _():
                wait_out(r)

            transpose(sb[r], db[r], W1)
            start_out(i, r)

            @pl.when(i + NRING1 < nb)
            def _():
                start_in(i + NRING1, r)

        def loop_body(i, carry):
            @pl.when(i < nb)
            def _():
                for r in range(NRING1):
                    @pl.when(i % NRING1 == r)
                    def _(r=r):
                        stage_step(i, r)

            return carry

        lax.fori_loop(0, NFULL // nw + 1, loop_body, 0)

        for r in range(NRING1):
            wait_out(r)

        @pl.when(wid == 2)
        def _():
            v0 = NFULL * W1
            pltpu.async_copy(
                tbl_hbm.at[:, pl.ds(v0, VTAIL)], st_v, si[0]).wait()

            def tbody(vl, colv):
                vals = [plsc.load_gather(st_v, [rowc[fc], colv])
                        for fc in range(D // 16)]
                off = vl * D
                for fc in range(D // 16):
                    dt_v[pl.ds(off + fc * 16, 16)] = vals[fc] * SCALE
                return colv + 1

            lax.fori_loop(0, VTAIL, tbody, jnp.zeros((16,), jnp.int32),
                          unroll=4)
            pltpu.async_copy(
                dt_v, tflat_hbm.at[pl.ds(v0 * D, VTAIL * D)], si[0]).wait()

    return k1(tbl_t, x_t)


@jax.jit
def _stage2(tbl_lin, idx2d):
    info = plsc.get_sparse_core_info()
    nw = info.num_cores * info.num_subcores
    mesh = plsc.VectorSubcoreMesh(core_axis_name="c", subcore_axis_name="s")
    n_half = NST * NBT * 2
    per_w = n_half // nw

    @functools.partial(
        pl.kernel,
        mesh=mesh,
        out_type=jax.ShapeDtypeStruct((S, 8, NBT, 1024), jnp.float32),
        scratch_types=(
            [pltpu.VMEM((4, 128), jnp.int32) for _ in range(NRING2)]
            + [pltpu.VMEM((512, D), jnp.float32) for _ in range(NRING2)]
            + [pltpu.VMEM((8, 1, 1024), jnp.float32) for _ in range(2)]
            + [pltpu.SemaphoreType.DMA for _ in range(2 * NRING2 + 2)]
        ),
        compiler_params=pltpu.CompilerParams(
            use_tc_tiling_on_sc=False, needs_layout_passes=False),
    )
    def k2(tbl_hbm, idx_hbm, out_hbm, *bufs):
        ib = bufs[:NRING2]
        rb = bufs[NRING2:2 * NRING2]
        t0, t1 = bufs[2 * NRING2:2 * NRING2 + 2]
        gi = bufs[2 * NRING2 + 2:3 * NRING2 + 2]
        gs = bufs[3 * NRING2 + 2:4 * NRING2 + 2]
        os0, os1 = bufs[4 * NRING2 + 2:]

        wid = lax.axis_index("s") * info.num_cores + lax.axis_index("c")

        def hu_of(j):
            return wid + j * nw

        def start_idx(j, r):
            pltpu.async_copy(idx_hbm.at[pl.ds(hu_of(j) * 4, 4)], ib[r],
                             gi[r])

        def wait_idx(r):
            pltpu.make_async_copy(
                idx_hbm.at[pl.ds(0, 4)], ib[r], gi[r]).wait()

        def start_gathers(r):
            for q in range(4):
                pltpu.async_copy(
                    tbl_hbm.at[ib[r].at[q]],
                    rb[r].at[pl.ds(q * 128, 128)], gs[r])

        def drain_gathers(r):
            for q in range(4):
                pltpu.make_async_copy(
                    tbl_hbm.at[ib[r].at[q]],
                    rb[r].at[pl.ds(q * 128, 128)], gs[r]).wait()

        def wait_out(tbuf, sem):
            pltpu.make_async_copy(
                tbuf, out_hbm.at[0, pl.ds(0, 8), pl.ds(0, 1)], sem).wait()

        iota = _iota16()

        def write_block(j, rbuf, sub_l, tbuf, sem):
            hu = hu_of(j)
            g = hu // 2
            h = hu % 2
            st = g // NBT
            bc = g % NBT
            s = st * 8 + h * 4 + sub_l
            row_base = [sub_l * 128 + c * 16 + iota for c in range(8)]

            def fbody(f, carry):
                fb = _bcast16(f)
                vals = [plsc.load_gather(rbuf, [row_base[c], fb])
                        for c in range(8)]
                ft = f // 8
                sub2 = f % 8
                off = sub2 * 128
                for c in range(8):
                    tbuf[ft, 0, pl.ds(off + c * 16, 16)] = vals[c]
                return carry

            lax.fori_loop(0, D, fbody, 0, unroll=2)
            pltpu.async_copy(
                tbuf, out_hbm.at[s, pl.ds(0, 8), pl.ds(bc, 1)], sem)

        def step(j, r):
            r2 = (r + 2) % NRING2
            drain_gathers(r)

            @pl.when(j + 2 < per_w)
            def _():
                wait_idx(r2)
                start_gathers(r2)

            @pl.when(j + 3 < per_w)
            def _():
                start_idx(j + 3, r)

            for sub_l in range(4):
                tb, osem = (t0, os0) if sub_l % 2 == 0 else (t1, os1)

                @pl.when(jnp.logical_or(j > 0, sub_l >= 2))
                def _():
                    wait_out(tb, osem)

                write_block(j, rb[r], sub_l, tb, osem)

        start_idx(0, 0)

        @pl.when(per_w > 1)
        def _():
            start_idx(1, 1)

        @pl.when(per_w > 2)
        def _():
            start_idx(2, 2)

        wait_idx(0)
        start_gathers(0)

        @pl.when(per_w > 1)
        def _():
            wait_idx(1)
            start_gathers(1)

        def loop_body(j, carry):
            for r in range(NRING2):
                @pl.when(j % NRING2 == r)
                def _(r=r):
                    step(j, r)

            return carry

        lax.fori_loop(0, per_w, loop_body, 0)

        wait_out(t0, os0)
        wait_out(t1, os1)

    return k2(tbl_lin, idx2d)


def kernel(x, table):
    x = x.astype(jnp.int32)
    tbl_t = table.T
    x_t = x.T
    tbl_flat, idx4 = _stage1(tbl_t, x_t)
    tbl_lin = tbl_flat.reshape(V, D)
    idx2d = idx4.reshape(NST * NBT * 8, 128)
    out4 = _stage2(tbl_lin, idx2d)
    out5 = out4.reshape(S, 8, NBT, 8, 128)
    return out5.transpose(2, 4, 0, 1, 3).reshape(B, S, D)


# speedup vs baseline: 2.0553x; 2.0553x over previous
import functools

import jax
import jax.numpy as jnp
from jax import lax
from jax.experimental import pallas as pl
from jax.experimental.pallas import tpu as pltpu
from jax.experimental.pallas import tpu_sc as plsc

D = 64
SCALE = 8.0
V = 1000000
B = 4096
S = 200

W1 = 128
NFULL = V // W1
VTAIL = V - NFULL * W1

NBT = B // 128
NST = S // 8

NRING1 = 4
NRING2 = 3


def _iota16():
    return lax.iota(jnp.int32, 16)


def _bcast16(v):
    return jnp.zeros((16,), jnp.int32) + v


@jax.jit
def _stage1(tbl_t, x_t):
    info = plsc.get_sparse_core_info()
    nw = info.num_cores * info.num_subcores
    mesh = plsc.VectorSubcoreMesh(core_axis_name="c", subcore_axis_name="s")
    n_xt = NST * NBT
    n_sup = n_xt // 4

    @functools.partial(
        pl.kernel,
        mesh=mesh,
        out_type=(
            jax.ShapeDtypeStruct((V * D,), jnp.float32),
            jax.ShapeDtypeStruct((n_xt, 8, 128), jnp.int32),
        ),
        scratch_types=(
            [pltpu.VMEM((D, W1), jnp.float32) for _ in range(NRING1)]
            + [pltpu.VMEM((W1 * D,), jnp.float32) for _ in range(NRING1)]
            + [
                pltpu.VMEM((W1 * 65,), jnp.float32),
                pltpu.VMEM((D, VTAIL), jnp.float32),
                pltpu.VMEM((VTAIL * D,), jnp.float32),
                pltpu.VMEM((8, 512), jnp.int32),
                pltpu.VMEM((8, 512), jnp.int32),
            ]
            + [pltpu.SemaphoreType.DMA for _ in range(2 * NRING1 + 2)]
        ),
        compiler_params=pltpu.CompilerParams(needs_layout_passes=False),
    )
    def k1(tbl_hbm, x_hbm, tflat_hbm, idx_hbm, *bufs):
        sb = bufs[:NRING1]
        db = bufs[NRING1:2 * NRING1]
        scr, st_v, dt_v, xa, xb = bufs[2 * NRING1:2 * NRING1 + 5]
        si = bufs[2 * NRING1 + 5:3 * NRING1 + 5]
        so = bufs[3 * NRING1 + 5:4 * NRING1 + 5]
        xsem, wsem = bufs[4 * NRING1 + 5:]

        wid = lax.axis_index("s") * info.num_cores + lax.axis_index("c")

        nb = NFULL // nw + jnp.where(wid < NFULL % nw, 1, 0)

        def blk_of(i):
            return wid + i * nw

        def start_in(i, r):
            pltpu.async_copy(
                tbl_hbm.at[:, pl.ds(blk_of(i) * W1, W1)], sb[r], si[r])

        def wait_in(r):
            pltpu.make_async_copy(
                tbl_hbm.at[:, pl.ds(0, W1)], sb[r], si[r]).wait()

        def start_out(i, r):
            pltpu.async_copy(
                db[r], tflat_hbm.at[pl.ds(blk_of(i) * W1 * D, W1 * D)],
                so[r])

        def wait_out(r):
            pltpu.make_async_copy(
                db[r], tflat_hbm.at[pl.ds(0, W1 * D)], so[r]).wait()

        for r in range(NRING1):
            @pl.when(nb > r)
            def _(r=r):
                start_in(r, r)

        n_x = n_sup // nw + jnp.where(wid < n_sup % nw, 1, 0)

        def sup_of(t):
            return wid + t * nw

        def x_read(t, buf):
            sup = sup_of(t)
            st = sup // (NBT // 4)
            bc4 = sup % (NBT // 4)
            pltpu.async_copy(
                x_hbm.at[pl.ds(st * 8, 8), pl.ds(bc4 * 512, 512)], buf, xsem)

        def x_step(t, buf):
            pltpu.make_async_copy(
                x_hbm.at[pl.ds(0, 8), pl.ds(0, 512)], buf, xsem).wait()
            tile0 = sup_of(t) * 4
            for k in range(4):
                pltpu.async_copy(
                    buf.at[:, pl.ds(k * 128, 128)],
                    idx_hbm.at[tile0 + k], wsem).wait()

            @pl.when(t + 2 < n_x)
            def _():
                x_read(t + 2, buf)

        x_read(0, xa)

        @pl.when(n_x > 1)
        def _():
            x_read(1, xb)

        def x_loop(t, carry):
            @pl.when(t < n_x)
            def _():
                @pl.when(t % 2 == 0)
                def _():
                    x_step(t, xa)

                @pl.when(t % 2 == 1)
                def _():
                    x_step(t, xb)

            return carry

        lax.fori_loop(0, n_sup // nw + 1, x_loop, 0)

        iota = _iota16()
        cbase = tuple((c * 16 + iota) * 65 for c in range(W1 // 16))

        def transpose(sbuf, dbuf, width):
            def p1(f, fv):
                vals = [sbuf[f, pl.ds(c * 16, 16)] * SCALE
                        for c in range(width // 16)]
                for c in range(width // 16):
                    plsc.store_scatter(scr, [cbase[c] + fv], vals[c])
                return fv + 1

            lax.fori_loop(0, D, p1, jnp.zeros((16,), jnp.int32), unroll=2)

            def p2(vl, carry):
                src_off = vl * 65
                dst_off = vl * D
                for fc in range(D // 16):
                    dbuf[pl.ds(dst_off + fc * 16, 16)] = (
                        scr[pl.ds(src_off + fc * 16, 16)])
                return carry

            lax.fori_loop(0, width, p2, 0, unroll=4)

        def stage_step(i, r):
            wait_in(r)

            @pl.when(i >= NRING1)
            def _():
                wait_out(r)

            transpose(sb[r], db[r], W1)
            start_out(i, r)

            @pl.when(i + NRING1 < nb)
            def _():
                start_in(i + NRING1, r)

        def loop_body(i, carry):
            @pl.when(i < nb)
            def _():
                for r in range(NRING1):
                    @pl.when(i % NRING1 == r)
                    def _(r=r):
                        stage_step(i, r)

            return carry

        lax.fori_loop(0, NFULL // nw + 1, loop_body, 0)

        for r in range(NRING1):
            wait_out(r)

        @pl.when(wid == 2)
        def _():
            v0 = NFULL * W1
            pltpu.async_copy(
                tbl_hbm.at[:, pl.ds(v0, VTAIL)], st_v, si[0]).wait()
            transpose(st_v, dt_v, VTAIL)
            pltpu.async_copy(
                dt_v, tflat_hbm.at[pl.ds(v0 * D, VTAIL * D)], si[0]).wait()

    return k1(tbl_t, x_t)


@jax.jit
def _stage2(tbl_lin, idx2d):
    info = plsc.get_sparse_core_info()
    nw = info.num_cores * info.num_subcores
    mesh = plsc.VectorSubcoreMesh(core_axis_name="c", subcore_axis_name="s")
    n_half = NST * NBT * 2
    per_w = n_half // nw

    @functools.partial(
        pl.kernel,
        mesh=mesh,
        out_type=jax.ShapeDtypeStruct((S, 8, NBT, 1024), jnp.float32),
        scratch_types=(
            [pltpu.VMEM((4, 128), jnp.int32) for _ in range(NRING2)]
            + [pltpu.VMEM((512, D), jnp.float32) for _ in range(NRING2)]
            + [pltpu.VMEM((8, 1, 1024), jnp.float32) for _ in range(2)]
            + [pltpu.VMEM((D * 129,), jnp.float32)]
            + [pltpu.SemaphoreType.DMA for _ in range(2 * NRING2 + 2)]
        ),
        compiler_params=pltpu.CompilerParams(
            use_tc_tiling_on_sc=False, needs_layout_passes=False),
    )
    def k2(tbl_hbm, idx_hbm, out_hbm, *bufs):
        ib = bufs[:NRING2]
        rb = bufs[NRING2:2 * NRING2]
        t0, t1 = bufs[2 * NRING2:2 * NRING2 + 2]
        scr = bufs[2 * NRING2 + 2]
        gi = bufs[2 * NRING2 + 3:3 * NRING2 + 3]
        gs = bufs[3 * NRING2 + 3:4 * NRING2 + 3]
        os0, os1 = bufs[4 * NRING2 + 3:]

        wid = lax.axis_index("s") * info.num_cores + lax.axis_index("c")

        def hu_of(j):
            return wid + j * nw

        def start_idx(j, r):
            pltpu.async_copy(idx_hbm.at[pl.ds(hu_of(j) * 4, 4)], ib[r],
                             gi[r])

        def wait_idx(r):
            pltpu.make_async_copy(
                idx_hbm.at[pl.ds(0, 4)], ib[r], gi[r]).wait()

        def start_gathers(r):
            for q in range(4):
                pltpu.async_copy(
                    tbl_hbm.at[ib[r].at[q]],
                    rb[r].at[pl.ds(q * 128, 128)], gs[r])

        def drain_gathers(r):
            for q in range(4):
                pltpu.make_async_copy(
                    tbl_hbm.at[ib[r].at[q]],
                    rb[r].at[pl.ds(q * 128, 128)], gs[r]).wait()

        def wait_out(tbuf, sem):
            pltpu.make_async_copy(
                tbuf, out_hbm.at[0, pl.ds(0, 8), pl.ds(0, 1)], sem).wait()

        iota = _iota16()

        fbase = tuple((fc * 16 + iota) * 129 for fc in range(D // 16))

        def write_block(j, rbuf, sub_l, tbuf, sem):
            hu = hu_of(j)
            g = hu // 2
            h = hu % 2
            st = g // NBT
            bc = g % NBT
            s = st * 8 + h * 4 + sub_l
            row0 = sub_l * 128

            def p1(l, lv):
                vals = [rbuf[row0 + l, pl.ds(fc * 16, 16)]
                        for fc in range(D // 16)]
                for fc in range(D // 16):
                    plsc.store_scatter(scr, [fbase[fc] + lv], vals[fc])
                return lv + 1

            lax.fori_loop(0, 128, p1, jnp.zeros((16,), jnp.int32),
                          unroll=2)

            def p2(ft, carry):
                def p2i(sub2, c2):
                    src0 = (ft * 8 + sub2) * 129
                    dst0 = sub2 * 128
                    for lc in range(8):
                        tbuf[ft, 0, pl.ds(dst0 + lc * 16, 16)] = (
                            scr[pl.ds(src0 + lc * 16, 16)])
                    return c2

                lax.fori_loop(0, 8, p2i, 0)
                return carry

            lax.fori_loop(0, 8, p2, 0)
            pltpu.async_copy(
                tbuf, out_hbm.at[s, pl.ds(0, 8), pl.ds(bc, 1)], sem)

        def step(j, r):
            r2 = (r + 2) % NRING2
            drain_gathers(r)

            @pl.when(j + 2 < per_w)
            def _():
                wait_idx(r2)
                start_gathers(r2)

            @pl.when(j + 3 < per_w)
            def _():
                start_idx(j + 3, r)

            for sub_l in range(4):
                tb, osem = (t0, os0) if sub_l % 2 == 0 else (t1, os1)

                @pl.when(jnp.logical_or(j > 0, sub_l >= 2))
                def _():
                    wait_out(tb, osem)

                write_block(j, rb[r], sub_l, tb, osem)

        start_idx(0, 0)

        @pl.when(per_w > 1)
        def _():
            start_idx(1, 1)

        @pl.when(per_w > 2)
        def _():
            start_idx(2, 2)

        wait_idx(0)
        start_gathers(0)

        @pl.when(per_w > 1)
        def _():
            wait_idx(1)
            start_gathers(1)

        def loop_body(j, carry):
            for r in range(NRING2):
                @pl.when(j % NRING2 == r)
                def _(r=r):
                    step(j, r)

            return carry

        lax.fori_loop(0, per_w, loop_body, 0)

        wait_out(t0, os0)
        wait_out(t1, os1)

    return k2(tbl_lin, idx2d)


def kernel(x, table):
    x = x.astype(jnp.int32)
    tbl_t = table.T
    x_t = x.T
    tbl_flat, idx4 = _stage1(tbl_t, x_t)
    tbl_lin = tbl_flat.reshape(V, D)
    idx2d = idx4.reshape(NST * NBT * 8, 128)
    out4 = _stage2(tbl_lin, idx2d)
    out5 = out4.reshape(S, 8, NBT, 8, 128)
    return out5.transpose(2, 4, 0, 1, 3).reshape(B, S, D)


# speedup vs baseline: 2.0669x; 1.0057x over previous
import functools

import jax
import jax.numpy as jnp
from jax import lax
from jax.experimental import pallas as pl
from jax.experimental.pallas import tpu as pltpu
from jax.experimental.pallas import tpu_sc as plsc

D = 64
SCALE = 8.0
V = 1000000
B = 4096
S = 200

W1 = 128
PAIR = 2 * W1
NPAIR = V // PAIR
VTAIL = V - NPAIR * PAIR

NBT = B // 128
NST = S // 8

NRING1 = 2
NRING2 = 4


def _iota16():
    return lax.iota(jnp.int32, 16)


@jax.jit
def _stage1(tbl_t, x_t):
    info = plsc.get_sparse_core_info()
    nw = info.num_cores * info.num_subcores
    mesh = plsc.VectorSubcoreMesh(core_axis_name="c", subcore_axis_name="s")
    n_xt = NST * NBT
    n_sup = n_xt // 4

    @functools.partial(
        pl.kernel,
        mesh=mesh,
        out_type=(
            jax.ShapeDtypeStruct((V * D,), jnp.float32),
            jax.ShapeDtypeStruct((n_xt, 8, 128), jnp.int32),
        ),
        scratch_types=(
            [pltpu.VMEM((2 * D, W1), jnp.float32) for _ in range(NRING1)]
            + [pltpu.VMEM((PAIR * D,), jnp.float32) for _ in range(NRING1)]
            + [
                pltpu.VMEM((W1 * 65,), jnp.float32),
                pltpu.VMEM((D, VTAIL), jnp.float32),
                pltpu.VMEM((VTAIL * D,), jnp.float32),
                pltpu.VMEM((8, 512), jnp.int32),
                pltpu.VMEM((8, 512), jnp.int32),
            ]
            + [pltpu.SemaphoreType.DMA for _ in range(2 * NRING1 + 2)]
        ),
        compiler_params=pltpu.CompilerParams(needs_layout_passes=False),
    )
    def k1(tbl_hbm, x_hbm, tflat_hbm, idx_hbm, *bufs):
        sb = bufs[:NRING1]
        db = bufs[NRING1:2 * NRING1]
        scr, st_v, dt_v, xa, xb = bufs[2 * NRING1:2 * NRING1 + 5]
        si = bufs[2 * NRING1 + 5:3 * NRING1 + 5]
        so = bufs[3 * NRING1 + 5:4 * NRING1 + 5]
        xsem, wsem = bufs[4 * NRING1 + 5:]

        wid = lax.axis_index("s") * info.num_cores + lax.axis_index("c")

        nb = NPAIR // nw + jnp.where(wid < NPAIR % nw, 1, 0)

        def pair_of(i):
            return wid + i * nw

        def start_in(i, r):
            v0 = pair_of(i) * PAIR
            for h in range(2):
                pltpu.async_copy(
                    tbl_hbm.at[:, pl.ds(v0 + h * W1, W1)],
                    sb[r].at[pl.ds(h * D, D)], si[r])

        def wait_in(r):
            for h in range(2):
                pltpu.make_async_copy(
                    tbl_hbm.at[:, pl.ds(0, W1)],
                    sb[r].at[pl.ds(h * D, D)], si[r]).wait()

        def start_out(i, r):
            pltpu.async_copy(
                db[r], tflat_hbm.at[pl.ds(pair_of(i) * PAIR * D, PAIR * D)],
                so[r])

        def wait_out(r):
            pltpu.make_async_copy(
                db[r], tflat_hbm.at[pl.ds(0, PAIR * D)], so[r]).wait()

        for r in range(NRING1):
            @pl.when(nb > r)
            def _(r=r):
                start_in(r, r)

        n_x = n_sup // nw + jnp.where(wid < n_sup % nw, 1, 0)

        def sup_of(t):
            return wid + t * nw

        def x_read(t, buf):
            sup = sup_of(t)
            st = sup // (NBT // 4)
            bc4 = sup % (NBT // 4)
            pltpu.async_copy(
                x_hbm.at[pl.ds(st * 8, 8), pl.ds(bc4 * 512, 512)], buf, xsem)

        def x_step(t, buf):
            pltpu.make_async_copy(
                x_hbm.at[pl.ds(0, 8), pl.ds(0, 512)], buf, xsem).wait()
            tile0 = sup_of(t) * 4
            for k in range(4):
                pltpu.async_copy(
                    buf.at[:, pl.ds(k * 128, 128)],
                    idx_hbm.at[tile0 + k], wsem).wait()

            @pl.when(t + 2 < n_x)
            def _():
                x_read(t + 2, buf)

        x_read(0, xa)

        @pl.when(n_x > 1)
        def _():
            x_read(1, xb)

        def x_loop(t, carry):
            @pl.when(t < n_x)
            def _():
                @pl.when(t % 2 == 0)
                def _():
                    x_step(t, xa)

                @pl.when(t % 2 == 1)
                def _():
                    x_step(t, xb)

            return carry

        lax.fori_loop(0, n_sup // nw + 1, x_loop, 0)

        iota = _iota16()
        cbase = tuple((c * 16 + iota) * 65 for c in range(W1 // 16))

        def transpose(sbuf, row0, dbuf, dst0, width):
            def p1(f, fv):
                vals = [sbuf[row0 + f, pl.ds(c * 16, 16)] * SCALE
                        for c in range(width // 16)]
                for c in range(width // 16):
                    plsc.store_scatter(scr, [cbase[c] + fv], vals[c])
                return fv + 1

            lax.fori_loop(0, D, p1, jnp.zeros((16,), jnp.int32), unroll=2)

            def p2(vl, carry):
                src_off = vl * 65
                dst_off = dst0 + vl * D
                for fc in range(D // 16):
                    dbuf[pl.ds(dst_off + fc * 16, 16)] = (
                        scr[pl.ds(src_off + fc * 16, 16)])
                return carry

            lax.fori_loop(0, width, p2, 0, unroll=4)

        def stage_step(i, r):
            wait_in(r)

            @pl.when(i >= NRING1)
            def _():
                wait_out(r)

            for h in range(2):
                transpose(sb[r], h * D, db[r], h * W1 * D, W1)
            start_out(i, r)

            @pl.when(i + NRING1 < nb)
            def _():
                start_in(i + NRING1, r)

        def loop_body(i, carry):
            @pl.when(i < nb)
            def _():
                for r in range(NRING1):
                    @pl.when(i % NRING1 == r)
                    def _(r=r):
                        stage_step(i, r)

            return carry

        lax.fori_loop(0, NPAIR // nw + 1, loop_body, 0)

        for r in range(NRING1):
            wait_out(r)

        @pl.when(wid == 2)
        def _():
            v0 = NPAIR * PAIR
            pltpu.async_copy(
                tbl_hbm.at[:, pl.ds(v0, VTAIL)], st_v, si[0]).wait()
            transpose(st_v, 0, dt_v, 0, VTAIL)
            pltpu.async_copy(
                dt_v, tflat_hbm.at[pl.ds(v0 * D, VTAIL * D)], si[0]).wait()

    return k1(tbl_t, x_t)


@jax.jit
def _stage2(tbl_lin, idx2d):
    info = plsc.get_sparse_core_info()
    nw = info.num_cores * info.num_subcores
    mesh = plsc.VectorSubcoreMesh(core_axis_name="c", subcore_axis_name="s")
    n_unit = NST * NBT * 4
    per_w = n_unit // nw

    @functools.partial(
        pl.kernel,
        mesh=mesh,
        out_type=jax.ShapeDtypeStruct((S, 8, NBT, 1024), jnp.float32),
        scratch_types=(
            [pltpu.VMEM((2, 128), jnp.int32) for _ in range(NRING2)]
            + [pltpu.VMEM((256, D), jnp.float32) for _ in range(NRING2)]
            + [pltpu.VMEM((8, 1, 1024), jnp.float32) for _ in range(2)]
            + [pltpu.VMEM((D * 129,), jnp.float32)]
            + [pltpu.SemaphoreType.DMA for _ in range(2 * NRING2 + 2)]
        ),
        compiler_params=pltpu.CompilerParams(
            use_tc_tiling_on_sc=False, needs_layout_passes=False),
    )
    def k2(tbl_hbm, idx_hbm, out_hbm, *bufs):
        ib = bufs[:NRING2]
        rb = bufs[NRING2:2 * NRING2]
        t0, t1 = bufs[2 * NRING2:2 * NRING2 + 2]
        scr = bufs[2 * NRING2 + 2]
        gi = bufs[2 * NRING2 + 3:3 * NRING2 + 3]
        gs = bufs[3 * NRING2 + 3:4 * NRING2 + 3]
        os0, os1 = bufs[4 * NRING2 + 3:]

        wid = lax.axis_index("s") * info.num_cores + lax.axis_index("c")

        def hu_of(j):
            return wid + j * nw

        def start_idx(j, r):
            pltpu.async_copy(idx_hbm.at[pl.ds(hu_of(j) * 2, 2)], ib[r],
                             gi[r])

        def wait_idx(r):
            pltpu.make_async_copy(
                idx_hbm.at[pl.ds(0, 2)], ib[r], gi[r]).wait()

        def start_gathers(r):
            for q in range(2):
                pltpu.async_copy(
                    tbl_hbm.at[ib[r].at[q]],
                    rb[r].at[pl.ds(q * 128, 128)], gs[r])

        def drain_gathers(r):
            for q in range(2):
                pltpu.make_async_copy(
                    tbl_hbm.at[ib[r].at[q]],
                    rb[r].at[pl.ds(q * 128, 128)], gs[r]).wait()

        def wait_out(tbuf, sem):
            pltpu.make_async_copy(
                tbuf, out_hbm.at[0, pl.ds(0, 8), pl.ds(0, 1)], sem).wait()

        iota = _iota16()
        fbase = tuple((fc * 16 + iota) * 129 for fc in range(D // 16))

        def write_block(j, rbuf, sub_l, tbuf, sem):
            hu = hu_of(j)
            g = hu // 4
            st = g // NBT
            bc = g % NBT
            s = st * 8 + (hu % 4) * 2 + sub_l
            row0 = sub_l * 128

            def p1(l, lv):
                vals = [rbuf[row0 + l, pl.ds(fc * 16, 16)]
                        for fc in range(D // 16)]
                for fc in range(D // 16):
                    plsc.store_scatter(scr, [fbase[fc] + lv], vals[fc])
                return lv + 1

            lax.fori_loop(0, 128, p1, jnp.zeros((16,), jnp.int32),
                          unroll=2)

            def p2(ft, carry):
                def p2i(sub2, c2):
                    src0 = (ft * 8 + sub2) * 129
                    dst0 = sub2 * 128
                    for lc in range(8):
                        tbuf[ft, 0, pl.ds(dst0 + lc * 16, 16)] = (
                            scr[pl.ds(src0 + lc * 16, 16)])
                    return c2

                lax.fori_loop(0, 8, p2i, 0)
                return carry

            lax.fori_loop(0, 8, p2, 0)
            pltpu.async_copy(
                tbuf, out_hbm.at[s, pl.ds(0, 8), pl.ds(bc, 1)], sem)

        def step(j, r):
            r3 = (r + 3) % NRING2
            drain_gathers(r)

            @pl.when(j + 3 < per_w)
            def _():
                wait_idx(r3)
                start_gathers(r3)

            @pl.when(j + 4 < per_w)
            def _():
                start_idx(j + 4, r)

            for sub_l in range(2):
                tb, osem = (t0, os0) if sub_l == 0 else (t1, os1)

                @pl.when(j > 0)
                def _():
                    wait_out(tb, osem)

                write_block(j, rb[r], sub_l, tb, osem)

        for r in range(NRING2):
            @pl.when(per_w > r)
            def _(r=r):
                start_idx(r, r)

        for r in range(3):
            @pl.when(per_w > r)
            def _(r=r):
                wait_idx(r)
                start_gathers(r)

        def loop_body(j, carry):
            for r in range(NRING2):
                @pl.when(j % NRING2 == r)
                def _(r=r):
                    step(j, r)

            return carry

        lax.fori_loop(0, per_w, loop_body, 0)

        wait_out(t0, os0)
        wait_out(t1, os1)

    return k2(tbl_lin, idx2d)


def kernel(x, table):
    x = x.astype(jnp.int32)
    tbl_t = table.T
    x_t = x.T
    tbl_flat, idx4 = _stage1(tbl_t, x_t)
    tbl_lin = tbl_flat.reshape(V, D)
    idx2d = idx4.reshape(NST * NBT * 8, 128)
    out4 = _stage2(tbl_lin, idx2d)
    out5 = out4.reshape(S, 8, NBT, 8, 128)
    return out5.transpose(2, 4, 0, 1, 3).reshape(B, S, D)
